# baseline (speedup 1.0000x reference)
import functools

import jax
import jax.numpy as jnp
from jax import lax
from jax.experimental import pallas as pl
from jax.experimental.pallas import tpu as pltpu
from jax.experimental.pallas import tpu_sc as plsc

B, D, V = 128, 512, 100000
CHUNK = 8192
NBLK = (V + CHUNK - 1) // CHUNK
NEG = -1e30

NW_ACT = 8
RPW = B // NW_ACT


def _tc_body(x_ref, wt_ref, b_ref, act_ref, wv_ref, bv_ref,
             lse_ref, ba_ref, ent_ref, val_ref,
             m_ref, s_ref, t_ref, a_ref):
    i = pl.program_id(0)

    @pl.when(i == 0)
    def _init():
        m_ref[...] = jnp.full((B, 1), NEG, dtype=jnp.float32)
        s_ref[...] = jnp.zeros((B, 1), dtype=jnp.float32)
        t_ref[...] = jnp.zeros((B, 1), dtype=jnp.float32)
        a_ref[...] = jnp.zeros((B, 1), dtype=jnp.float32)
        val_ref[...] = (jnp.sum(x_ref[...] * wv_ref[...], axis=1, keepdims=True)
                        + bv_ref[...])

    L = lax.dot_general(x_ref[...], wt_ref[...],
                        (((1,), (1,)), ((), ())),
                        preferred_element_type=jnp.float32) + b_ref[...]

    cols = jax.lax.broadcasted_iota(jnp.int32, (B, CHUNK), 1) + i * CHUNK
    L = jnp.where(cols < V, L, NEG)

    mc = jnp.max(L, axis=1, keepdims=True)
    m_old = m_ref[...]
    m_new = jnp.maximum(m_old, mc)
    alpha = jnp.exp(m_old - m_new)
    e = jnp.exp(L - m_new)
    s_ref[...] = s_ref[...] * alpha + jnp.sum(e, axis=1, keepdims=True)
    t_ref[...] = t_ref[...] * alpha + jnp.sum(L * e, axis=1, keepdims=True)
    m_ref[...] = m_new

    onehot = cols == act_ref[...]
    b_bc = jnp.broadcast_to(b_ref[...], (B, CHUNK))
    a_ref[...] += jnp.sum(jnp.where(onehot, b_bc, 0.0), axis=1, keepdims=True)

    @pl.when(i == NBLK - 1)
    def _fin():
        lse = m_ref[...] + jnp.log(s_ref[...])
        lse_ref[...] = lse
        ba_ref[...] = a_ref[...]
        ent_ref[...] = lse - t_ref[...] / s_ref[...]


_sc_mesh = plsc.VectorSubcoreMesh(core_axis_name="c", subcore_axis_name="s")


@functools.partial(
    pl.kernel,
    out_type=jax.ShapeDtypeStruct((B,), jnp.float32),
    mesh=_sc_mesh,
    scratch_types=[
        pltpu.VMEM((RPW,), jnp.int32),
        pltpu.VMEM((RPW, D), jnp.float32),
        pltpu.VMEM((RPW, D), jnp.float32),
        pltpu.VMEM((16,), jnp.float32),
        pltpu.SemaphoreType.DMA,
    ],
)
def _sc_action_dot(wt_hbm, x_hbm, act_hbm, out_hbm,
                   idx_v, rows_v, xrows_v, out_v, sem):
    wid = lax.axis_index("s") * 2 + lax.axis_index("c")

    @pl.when(wid < NW_ACT)
    def _work():
        base = wid * RPW
        pltpu.sync_copy(act_hbm.at[pl.ds(base, RPW)], idx_v)
        pltpu.async_copy(wt_hbm.at[idx_v], rows_v, sem).wait()
        pltpu.sync_copy(x_hbm.at[pl.ds(base, RPW)], xrows_v)

        lane = lax.iota(jnp.int32, 16)
        res = jnp.zeros((16,), jnp.float32)
        for r in range(RPW):
            acc = jnp.zeros((16,), jnp.float32)
            for k in range(D // 16):
                acc = acc + (xrows_v[r, pl.ds(k * 16, 16)]
                             * rows_v[r, pl.ds(k * 16, 16)])
            for sh in (8, 4, 2, 1):
                rot = lax.gather(
                    acc, ((lane + sh) & 15)[:, None],
                    lax.GatherDimensionNumbers(
                        offset_dims=(), collapsed_slice_dims=(0,),
                        start_index_map=(0,)),
                    (1,), mode=lax.GatherScatterMode.PROMISE_IN_BOUNDS)
                acc = acc + rot
            res = jnp.where(lane == r, acc, res)
        out_v[...] = res
        pltpu.sync_copy(out_v, out_hbm.at[pl.ds(base, RPW)])


def kernel(x, W_pi, b_pi, W_v, b_v, action):
    act_i32 = action.astype(jnp.int32)
    act2d = act_i32.reshape(B, 1)
    b2d = b_pi.reshape(1, V)
    wv2d = W_v.reshape(1, D)
    bv2d = b_v.reshape(1, 1)
    Wt = W_pi.T

    a_raw = _sc_action_dot(Wt, x, act_i32)

    lse, b_at, ent, val = pl.pallas_call(
        _tc_body,
        grid=(NBLK,),
        in_specs=[
            pl.BlockSpec((B, D), lambda i: (0, 0)),
            pl.BlockSpec((CHUNK, D), lambda i: (i, 0)),
            pl.BlockSpec((1, CHUNK), lambda i: (0, i)),
            pl.BlockSpec((B, 1), lambda i: (0, 0)),
            pl.BlockSpec((1, D), lambda i: (0, 0)),
            pl.BlockSpec((1, 1), lambda i: (0, 0)),
        ],
        out_specs=[
            pl.BlockSpec((B, 1), lambda i: (0, 0)),
            pl.BlockSpec((B, 1), lambda i: (0, 0)),
            pl.BlockSpec((B, 1), lambda i: (0, 0)),
            pl.BlockSpec((B, 1), lambda i: (0, 0)),
        ],
        out_shape=[
            jax.ShapeDtypeStruct((B, 1), jnp.float32),
            jax.ShapeDtypeStruct((B, 1), jnp.float32),
            jax.ShapeDtypeStruct((B, 1), jnp.float32),
            jax.ShapeDtypeStruct((B, 1), jnp.float32),
        ],
        scratch_shapes=[
            pltpu.VMEM((B, 1), jnp.float32),
            pltpu.VMEM((B, 1), jnp.float32),
            pltpu.VMEM((B, 1), jnp.float32),
            pltpu.VMEM((B, 1), jnp.float32),
        ],
        compiler_params=pltpu.CompilerParams(
            dimension_semantics=("arbitrary",),
        ),
    )(x, Wt, b2d, act2d, wv2d, bv2d)

    logprob = a_raw + b_at.reshape(B) - lse.reshape(B)
    return (action, logprob, ent.reshape(B), val)


# speedup vs baseline: 1.1389x; 1.1389x over previous
import jax
import jax.numpy as jnp
from jax import lax
from jax.experimental import pallas as pl
from jax.experimental.pallas import tpu as pltpu

B, D, V = 128, 512, 100000
CHUNK = 4096
NBLK = (V + CHUNK - 1) // CHUNK
NEG = -1e30


def _body(x_ref, wt_ref, b_ref, act_ref, wv_ref, bv_ref,
          lp_ref, ent_ref, val_ref,
          m_ref, s_ref, t_ref, a_ref):
    i = pl.program_id(0)

    @pl.when(i == 0)
    def _init():
        m_ref[...] = jnp.full((B, 1), NEG, dtype=jnp.float32)
        s_ref[...] = jnp.zeros((B, 1), dtype=jnp.float32)
        t_ref[...] = jnp.zeros((B, 1), dtype=jnp.float32)
        a_ref[...] = jnp.zeros((B, 1), dtype=jnp.float32)
        val_ref[...] = (jnp.sum(x_ref[...] * wv_ref[...], axis=1, keepdims=True)
                        + bv_ref[...])

    L = lax.dot_general(x_ref[...], wt_ref[...],
                        (((1,), (1,)), ((), ())),
                        preferred_element_type=jnp.float32) + b_ref[...]

    cols = jax.lax.broadcasted_iota(jnp.int32, (B, CHUNK), 1) + i * CHUNK
    L = jnp.where(cols < V, L, NEG)

    mc = jnp.max(L, axis=1, keepdims=True)
    m_old = m_ref[...]
    m_new = jnp.maximum(m_old, mc)
    alpha = jnp.exp(m_old - m_new)
    e = jnp.exp(L - m_new)
    s_ref[...] = s_ref[...] * alpha + jnp.sum(e, axis=1, keepdims=True)
    t_ref[...] = t_ref[...] * alpha + jnp.sum(L * e, axis=1, keepdims=True)
    m_ref[...] = m_new

    a_ref[...] += jnp.sum(jnp.where(cols == act_ref[...], L, 0.0),
                          axis=1, keepdims=True)

    @pl.when(i == NBLK - 1)
    def _fin():
        lse = m_ref[...] + jnp.log(s_ref[...])
        lp_ref[...] = a_ref[...] - lse
        ent_ref[...] = lse - t_ref[...] / s_ref[...]


def kernel(x, W_pi, b_pi, W_v, b_v, action):
    act2d = action.astype(jnp.int32).reshape(B, 1)
    b2d = b_pi.reshape(1, V)
    wv2d = W_v.reshape(1, D)
    bv2d = b_v.reshape(1, 1)
    Wt = W_pi.T

    lp, ent, val = pl.pallas_call(
        _body,
        grid=(NBLK,),
        in_specs=[
            pl.BlockSpec((B, D), lambda i: (0, 0)),
            pl.BlockSpec((CHUNK, D), lambda i: (i, 0)),
            pl.BlockSpec((1, CHUNK), lambda i: (0, i)),
            pl.BlockSpec((B, 1), lambda i: (0, 0)),
            pl.BlockSpec((1, D), lambda i: (0, 0)),
            pl.BlockSpec((1, 1), lambda i: (0, 0)),
        ],
        out_specs=[
            pl.BlockSpec((B, 1), lambda i: (0, 0)),
            pl.BlockSpec((B, 1), lambda i: (0, 0)),
            pl.BlockSpec((B, 1), lambda i: (0, 0)),
        ],
        out_shape=[
            jax.ShapeDtypeStruct((B, 1), jnp.float32),
            jax.ShapeDtypeStruct((B, 1), jnp.float32),
            jax.ShapeDtypeStruct((B, 1), jnp.float32),
        ],
        scratch_shapes=[
            pltpu.VMEM((B, 1), jnp.float32),
            pltpu.VMEM((B, 1), jnp.float32),
            pltpu.VMEM((B, 1), jnp.float32),
            pltpu.VMEM((B, 1), jnp.float32),
        ],
        compiler_params=pltpu.CompilerParams(
            dimension_semantics=("arbitrary",),
        ),
    )(x, Wt, b2d, act2d, wv2d, bv2d)

    return (action, lp.reshape(B), ent.reshape(B), val)


# speedup vs baseline: 1.1549x; 1.0140x over previous
import jax
import jax.numpy as jnp
from jax import lax
from jax.experimental import pallas as pl
from jax.experimental.pallas import tpu as pltpu

B, D, V = 128, 512, 100000
CHUNK = 12288
NBLK = (V + CHUNK - 1) // CHUNK
NEG = -1e30


def _body(x_ref, wt_ref, b_ref, act_ref, wv_ref, bv_ref,
          lp_ref, ent_ref, val_ref,
          m_ref, s_ref, t_ref, a_ref):
    i = pl.program_id(0)

    @pl.when(i == 0)
    def _init():
        m_ref[...] = jnp.full((B, 1), NEG, dtype=jnp.float32)
        s_ref[...] = jnp.zeros((B, 1), dtype=jnp.float32)
        t_ref[...] = jnp.zeros((B, 1), dtype=jnp.float32)
        a_ref[...] = jnp.zeros((B, 1), dtype=jnp.float32)
        val_ref[...] = (jnp.sum(x_ref[...] * wv_ref[...], axis=1, keepdims=True)
                        + bv_ref[...])

    L = lax.dot_general(x_ref[...], wt_ref[...],
                        (((1,), (1,)), ((), ())),
                        preferred_element_type=jnp.float32) + b_ref[...]

    cols = jax.lax.broadcasted_iota(jnp.int32, (B, CHUNK), 1) + i * CHUNK
    L = jnp.where(cols < V, L, NEG)

    mc = jnp.max(L, axis=1, keepdims=True)
    m_old = m_ref[...]
    m_new = jnp.maximum(m_old, mc)
    alpha = jnp.exp(m_old - m_new)
    e = jnp.exp(L - m_new)
    s_ref[...] = s_ref[...] * alpha + jnp.sum(e, axis=1, keepdims=True)
    t_ref[...] = t_ref[...] * alpha + jnp.sum(L * e, axis=1, keepdims=True)
    m_ref[...] = m_new

    a_ref[...] += jnp.sum(jnp.where(cols == act_ref[...], L, 0.0),
                          axis=1, keepdims=True)

    @pl.when(i == NBLK - 1)
    def _fin():
        lse = m_ref[...] + jnp.log(s_ref[...])
        lp_ref[...] = a_ref[...] - lse
        ent_ref[...] = lse - t_ref[...] / s_ref[...]


def kernel(x, W_pi, b_pi, W_v, b_v, action):
    act2d = action.astype(jnp.int32).reshape(B, 1)
    b2d = b_pi.reshape(1, V)
    wv2d = W_v.reshape(1, D)
    bv2d = b_v.reshape(1, 1)
    Wt = W_pi.T

    lp, ent, val = pl.pallas_call(
        _body,
        grid=(NBLK,),
        in_specs=[
            pl.BlockSpec((B, D), lambda i: (0, 0)),
            pl.BlockSpec((CHUNK, D), lambda i: (i, 0)),
            pl.BlockSpec((1, CHUNK), lambda i: (0, i)),
            pl.BlockSpec((B, 1), lambda i: (0, 0)),
            pl.BlockSpec((1, D), lambda i: (0, 0)),
            pl.BlockSpec((1, 1), lambda i: (0, 0)),
        ],
        out_specs=[
            pl.BlockSpec((B, 1), lambda i: (0, 0)),
            pl.BlockSpec((B, 1), lambda i: (0, 0)),
            pl.BlockSpec((B, 1), lambda i: (0, 0)),
        ],
        out_shape=[
            jax.ShapeDtypeStruct((B, 1), jnp.float32),
            jax.ShapeDtypeStruct((B, 1), jnp.float32),
            jax.ShapeDtypeStruct((B, 1), jnp.float32),
        ],
        scratch_shapes=[
            pltpu.VMEM((B, 1), jnp.float32),
            pltpu.VMEM((B, 1), jnp.float32),
            pltpu.VMEM((B, 1), jnp.float32),
            pltpu.VMEM((B, 1), jnp.float32),
        ],
        compiler_params=pltpu.CompilerParams(
            dimension_semantics=("arbitrary",),
        ),
    )(x, Wt, b2d, act2d, wv2d, bv2d)

    return (action, lp.reshape(B), ent.reshape(B), val)


# speedup vs baseline: 1.2360x; 1.0702x over previous
import jax
import jax.numpy as jnp
from jax import lax
from jax.experimental import pallas as pl
from jax.experimental.pallas import tpu as pltpu

B, D, V = 128, 512, 100000
CHUNK = 10240
NBLK = (V + CHUNK - 1) // CHUNK
NEG = -1e30


def _body(x_ref, wt_ref, b_ref, act_ref, wv_ref, bv_ref,
          lp_ref, ent_ref, val_ref,
          m_ref, s_ref, t_ref, a_ref):
    i = pl.program_id(0)

    @pl.when(i == 0)
    def _init():
        m_ref[...] = jnp.full((B, 1), NEG, dtype=jnp.float32)
        s_ref[...] = jnp.zeros((B, 1), dtype=jnp.float32)
        t_ref[...] = jnp.zeros((B, 1), dtype=jnp.float32)
        a_ref[...] = jnp.zeros((B, 1), dtype=jnp.float32)
        val_ref[...] = (jnp.sum(x_ref[...] * wv_ref[...], axis=1, keepdims=True)
                        + bv_ref[...])

    L = lax.dot_general(x_ref[...], wt_ref[...],
                        (((1,), (1,)), ((), ())),
                        preferred_element_type=jnp.float32) + b_ref[...]

    cols = jax.lax.broadcasted_iota(jnp.int32, (B, CHUNK), 1) + i * CHUNK
    L = jnp.where(cols < V, L, NEG)

    mc = jnp.max(L, axis=1, keepdims=True)
    m_old = m_ref[...]
    m_new = jnp.maximum(m_old, mc)
    alpha = jnp.exp(m_old - m_new)
    e = jnp.exp(L - m_new)
    s_ref[...] = s_ref[...] * alpha + jnp.sum(e, axis=1, keepdims=True)
    t_ref[...] = t_ref[...] * alpha + jnp.sum(L * e, axis=1, keepdims=True)
    m_ref[...] = m_new

    a_ref[...] += jnp.sum(jnp.where(cols == act_ref[...], L, 0.0),
                          axis=1, keepdims=True)

    @pl.when(i == NBLK - 1)
    def _fin():
        lse = m_ref[...] + jnp.log(s_ref[...])
        lp_ref[...] = a_ref[...] - lse
        ent_ref[...] = lse - t_ref[...] / s_ref[...]


def kernel(x, W_pi, b_pi, W_v, b_v, action):
    act2d = action.astype(jnp.int32).reshape(B, 1)
    b2d = b_pi.reshape(1, V)
    wv2d = W_v.reshape(1, D)
    bv2d = b_v.reshape(1, 1)
    Wt = W_pi.T

    lp, ent, val = pl.pallas_call(
        _body,
        grid=(NBLK,),
        in_specs=[
            pl.BlockSpec((B, D), lambda i: (0, 0)),
            pl.BlockSpec((CHUNK, D), lambda i: (i, 0)),
            pl.BlockSpec((1, CHUNK), lambda i: (0, i)),
            pl.BlockSpec((B, 1), lambda i: (0, 0)),
            pl.BlockSpec((1, D), lambda i: (0, 0)),
            pl.BlockSpec((1, 1), lambda i: (0, 0)),
        ],
        out_specs=[
            pl.BlockSpec((B, 1), lambda i: (0, 0)),
            pl.BlockSpec((B, 1), lambda i: (0, 0)),
            pl.BlockSpec((B, 1), lambda i: (0, 0)),
        ],
        out_shape=[
            jax.ShapeDtypeStruct((B, 1), jnp.float32),
            jax.ShapeDtypeStruct((B, 1), jnp.float32),
            jax.ShapeDtypeStruct((B, 1), jnp.float32),
        ],
        scratch_shapes=[
            pltpu.VMEM((B, 1), jnp.float32),
            pltpu.VMEM((B, 1), jnp.float32),
            pltpu.VMEM((B, 1), jnp.float32),
            pltpu.VMEM((B, 1), jnp.float32),
        ],
        compiler_params=pltpu.CompilerParams(
            dimension_semantics=("arbitrary",),
        ),
    )(x, Wt, b2d, act2d, wv2d, bv2d)

    return (action, lp.reshape(B), ent.reshape(B), val)
